# Initial kernel scaffold; baseline (speedup 1.0000x reference)
import jax
import jax.numpy as jnp
from jax import lax
from jax.experimental import pallas as pl
from jax.experimental.pallas import tpu as pltpu
from jax.experimental.pallas import tpu_sc as plsc

N = 10000
D = 128
H = 128
C = 70
E = 320000

NC, NS = 2, 16
CH = 128
NCHUNK = 80
EPW = CH * NCHUNK
EP = EPW * NC * NS
NR = 10016
RPT = NR // NS
NBUF = 4
DEGW = 16

_mesh = plsc.VectorSubcoreMesh(
    core_axis_name="c", subcore_axis_name="s", num_cores=NC, num_subcores=NS
)


def _sc_agg_body(g_hbm, src_hbm, dst_hbm, zeros_hbm, out_hbm,
                 src_v, dst_v, bufs, accum, gsem, ssem):
    cid = lax.axis_index("c")
    sid = lax.axis_index("s")
    pltpu.sync_copy(src_hbm.at[cid, sid], src_v)
    pltpu.sync_copy(dst_hbm.at[cid, sid], dst_v)
    pltpu.sync_copy(zeros_hbm.at[pl.ds(sid * RPT, RPT)],
                    accum.at[pl.ds(sid * RPT, RPT)])
    plsc.subcore_barrier()

    def group(gi, carry):
        gds = []
        for b in range(NBUF):
            j = gi * NBUF + b
            gds.append(pltpu.async_copy(g_hbm.at[src_v.at[j]], bufs.at[b], gsem))
        for d in gds:
            d.wait()
        sds = []
        for b in range(NBUF):
            j = gi * NBUF + b
            sds.append(
                pltpu.async_copy(bufs.at[b], accum.at[dst_v.at[j]], ssem,
                                 add=True))
        for d in sds:
            d.wait()
        return carry

    lax.fori_loop(0, NCHUNK // NBUF, group, 0)
    plsc.subcore_barrier()
    pltpu.sync_copy(accum.at[pl.ds(sid * RPT, RPT)],
                    out_hbm.at[cid, pl.ds(sid * RPT, RPT)])


_sc_agg = pl.kernel(
    _sc_agg_body,
    out_type=jax.ShapeDtypeStruct((NC, NR, D), jnp.float32),
    mesh=_mesh,
    scratch_types=[
        pltpu.VMEM((NCHUNK, 1, CH), jnp.int32),
        pltpu.VMEM((NCHUNK, 1, CH), jnp.int32),
        pltpu.VMEM((NBUF, CH, D), jnp.float32),
        pltpu.VMEM_SHARED((NR, D), jnp.float32),
        pltpu.SemaphoreType.DMA,
        pltpu.SemaphoreType.DMA,
    ],
)


def _sc_deg_body(dst_hbm, zeros_hbm, ones_hbm, out_hbm,
                 dst_v, ones_v, accum, ssem):
    cid = lax.axis_index("c")
    sid = lax.axis_index("s")
    pltpu.sync_copy(dst_hbm.at[cid, sid], dst_v)
    pltpu.sync_copy(ones_hbm, ones_v)
    pltpu.sync_copy(zeros_hbm.at[pl.ds(sid * RPT, RPT)],
                    accum.at[pl.ds(sid * RPT, RPT)])
    plsc.subcore_barrier()

    def group(gi, carry):
        sds = []
        for b in range(NBUF):
            j = gi * NBUF + b
            sds.append(
                pltpu.async_copy(ones_v, accum.at[dst_v.at[j]], ssem,
                                 add=True))
        for d in sds:
            d.wait()
        return carry

    lax.fori_loop(0, NCHUNK // NBUF, group, 0)
    plsc.subcore_barrier()
    pltpu.sync_copy(accum.at[pl.ds(sid * RPT, RPT)],
                    out_hbm.at[cid, pl.ds(sid * RPT, RPT)])


_sc_deg = pl.kernel(
    _sc_deg_body,
    out_type=jax.ShapeDtypeStruct((NC, NR, DEGW), jnp.float32),
    mesh=_mesh,
    scratch_types=[
        pltpu.VMEM((NCHUNK, 1, CH), jnp.int32),
        pltpu.VMEM((CH, DEGW), jnp.float32),
        pltpu.VMEM_SHARED((NR, DEGW), jnp.float32),
        pltpu.SemaphoreType.DMA,
    ],
)


def _tc1_body(x_ref, w1_ref, degp_ref, g1_ref, dinv_ref):
    degp = degp_ref[...]
    deg = degp[0, :N, 0:1] + degp[1, :N, 0:1] + 1.0
    dinv = lax.rsqrt(deg)
    h1 = jnp.dot(x_ref[...], w1_ref[...], preferred_element_type=jnp.float32)
    g1_ref[...] = h1 * dinv
    dinv_ref[...] = dinv


_tc1 = pl.pallas_call(
    _tc1_body,
    out_shape=(
        jax.ShapeDtypeStruct((N, H), jnp.float32),
        jax.ShapeDtypeStruct((N, 1), jnp.float32),
    ),
)


def _tc2_body(aggp_ref, g1_ref, dinv_ref, b1_ref, gamma_ref, beta_ref,
              w2_ref, g2_ref):
    aggp = aggp_ref[...]
    agg = aggp[0, :N, :] + aggp[1, :N, :]
    dinv = dinv_ref[...]
    c1 = dinv * (agg + g1_ref[...]) + b1_ref[...]
    mean = jnp.mean(c1, axis=0, keepdims=True)
    var = jnp.mean((c1 - mean) * (c1 - mean), axis=0, keepdims=True)
    hb = (c1 - mean) * lax.rsqrt(var + 1e-5) * gamma_ref[...] + beta_ref[...]
    hr = jnp.maximum(hb, 0.0)
    t2 = jnp.dot(hr, w2_ref[...], preferred_element_type=jnp.float32)
    g2_ref[...] = t2 * dinv


_tc2 = pl.pallas_call(
    _tc2_body,
    out_shape=jax.ShapeDtypeStruct((N, H), jnp.float32),
)


def _tc3_body(aggp_ref, g2_ref, dinv_ref, b2_ref, wc_ref, bc_ref, out_ref):
    aggp = aggp_ref[...]
    agg = aggp[0, :N, :] + aggp[1, :N, :]
    dinv = dinv_ref[...]
    c2 = dinv * (agg + g2_ref[...]) + b2_ref[...]
    out_ref[...] = (
        jnp.dot(c2, wc_ref[...], preferred_element_type=jnp.float32)
        + bc_ref[...])


_tc3 = pl.pallas_call(
    _tc3_body,
    out_shape=jax.ShapeDtypeStruct((N, C), jnp.float32),
)


def kernel(x, edge_index, W1, b1, gamma1, beta1, W2, b2, Wc, bc):
    src = edge_index[0]
    dst = edge_index[1]
    pad = EP - E
    srcp = jnp.concatenate([src, jnp.zeros((pad,), jnp.int32)])
    dstp = jnp.concatenate([dst, jnp.full((pad,), N, jnp.int32)])
    srcp = srcp.reshape(NC, NS, NCHUNK, 1, CH)
    dstp = dstp.reshape(NC, NS, NCHUNK, 1, CH)

    zeros_deg = jnp.zeros((NR, DEGW), jnp.float32)
    ones_ch = jnp.ones((CH, DEGW), jnp.float32)
    zeros_f = jnp.zeros((NR, D), jnp.float32)

    degp = _sc_deg(dstp, zeros_deg, ones_ch)
    g1, dinv = _tc1(x, W1, degp)
    agg1 = _sc_agg(g1, srcp, dstp, zeros_f)
    g2 = _tc2(agg1, g1, dinv, b1.reshape(1, H), gamma1.reshape(1, H),
              beta1.reshape(1, H), W2)
    agg2 = _sc_agg(g2, srcp, dstp, zeros_f)
    out = _tc3(agg2, g2, dinv, b2.reshape(1, H), Wc, bc.reshape(1, C))
    return out


# speedup vs baseline: 6.7787x; 6.7787x over previous
import jax
import jax.numpy as jnp
from jax import lax
from jax.experimental import pallas as pl
from jax.experimental.pallas import tpu as pltpu
from jax.experimental.pallas import tpu_sc as plsc

N = 10000
D = 128
H = 128
C = 70
E = 320000

NC, NS = 2, 16
CH = 128
NCHUNK = 80
EP = CH * NCHUNK * NC * NS
NR = 10112
RPT = NR // NS
NBUF = 2
IBLK = 16
DEGW = 128
_RCHUNKS = [(k * CH, min(CH, RPT - k * CH)) for k in range((RPT + CH - 1) // CH)]

_mesh = plsc.VectorSubcoreMesh(
    core_axis_name="c", subcore_axis_name="s", num_cores=NC, num_subcores=NS
)


def _sc_agg_body(g_hbm, src_hbm, dst_hbm, zeros_hbm, out_hbm,
                 src_v, dst_v, bufs, accum, gsem0, gsem1, ssem0, ssem1):
    cid = lax.axis_index("c")
    sid = lax.axis_index("s")
    base = sid * RPT
    gsems = (gsem0, gsem1)
    ssems = (ssem0, ssem1)
    pltpu.sync_copy(zeros_hbm, bufs.at[0])
    for off, rows in _RCHUNKS:
        pltpu.sync_copy(bufs.at[0].at[pl.ds(0, rows)],
                        accum.at[pl.ds(base + off, rows)])
    plsc.subcore_barrier()

    def block(bi, carry):
        pltpu.sync_copy(src_hbm.at[cid, sid].at[pl.ds(bi * IBLK, IBLK)],
                        src_v)
        pltpu.sync_copy(dst_hbm.at[cid, sid].at[pl.ds(bi * IBLK, IBLK)],
                        dst_v)
        for b in range(NBUF):
            pltpu.async_copy(g_hbm.at[src_v.at[b]], bufs.at[b], gsems[b])
        for k in range(IBLK):
            b = k % NBUF
            pltpu.make_async_copy(g_hbm.at[src_v.at[k]], bufs.at[b],
                                  gsems[b]).wait()
            pltpu.async_copy(bufs.at[b], accum.at[dst_v.at[k]], ssems[b],
                             add=True)
            pltpu.make_async_copy(bufs.at[b], accum.at[dst_v.at[k]],
                                  ssems[b]).wait()
            if k + NBUF < IBLK:
                pltpu.async_copy(g_hbm.at[src_v.at[k + NBUF]], bufs.at[b],
                                 gsems[b])
        return carry

    lax.fori_loop(0, NCHUNK // IBLK, block, 0)
    plsc.subcore_barrier()
    for off, rows in _RCHUNKS:
        pltpu.sync_copy(accum.at[pl.ds(base + off, rows)],
                        bufs.at[0].at[pl.ds(0, rows)])
        pltpu.sync_copy(bufs.at[0].at[pl.ds(0, rows)],
                        out_hbm.at[cid, pl.ds(base + off, rows)])


_sc_agg = pl.kernel(
    _sc_agg_body,
    out_type=jax.ShapeDtypeStruct((NC, NR, D), jnp.float32),
    mesh=_mesh,
    scratch_types=[
        pltpu.VMEM((IBLK, CH), jnp.int32),
        pltpu.VMEM((IBLK, CH), jnp.int32),
        pltpu.VMEM((NBUF, CH, D), jnp.float32),
        pltpu.VMEM_SHARED((NR, D), jnp.float32),
        pltpu.SemaphoreType.DMA,
        pltpu.SemaphoreType.DMA,
        pltpu.SemaphoreType.DMA,
        pltpu.SemaphoreType.DMA,
    ],
)


def _sc_deg_body(dst_hbm, zeros_hbm, ones_hbm, out_hbm,
                 dst_v, ones_v, accum, ssem):
    cid = lax.axis_index("c")
    sid = lax.axis_index("s")
    base = sid * RPT
    pltpu.sync_copy(dst_hbm.at[cid, sid], dst_v)
    pltpu.sync_copy(zeros_hbm, ones_v)
    for off, rows in _RCHUNKS:
        pltpu.sync_copy(ones_v.at[pl.ds(0, rows)],
                        accum.at[pl.ds(base + off, rows)])
    pltpu.sync_copy(ones_hbm, ones_v)
    plsc.subcore_barrier()

    def group(gi, carry):
        sds = []
        for b in range(NBUF):
            j = gi * NBUF + b
            sds.append(
                pltpu.async_copy(ones_v, accum.at[dst_v.at[j]], ssem,
                                 add=True))
        for d in sds:
            d.wait()
        return carry

    lax.fori_loop(0, NCHUNK // NBUF, group, 0)
    plsc.subcore_barrier()
    for off, rows in _RCHUNKS:
        pltpu.sync_copy(accum.at[pl.ds(base + off, rows)],
                        ones_v.at[pl.ds(0, rows)])
        pltpu.sync_copy(ones_v.at[pl.ds(0, rows)],
                        out_hbm.at[cid, pl.ds(base + off, rows)])


_sc_deg = pl.kernel(
    _sc_deg_body,
    out_type=jax.ShapeDtypeStruct((NC, NR, DEGW), jnp.float32),
    mesh=_mesh,
    scratch_types=[
        pltpu.VMEM((NCHUNK, CH), jnp.int32),
        pltpu.VMEM((CH, DEGW), jnp.float32),
        pltpu.VMEM_SHARED((NR, DEGW), jnp.float32),
        pltpu.SemaphoreType.DMA,
    ],
)


def _tc1_body(x_ref, w1_ref, degp_ref, g1_ref, dinv_ref):
    degp = degp_ref[...]
    deg = degp[0, :N, 0:1] + degp[1, :N, 0:1] + 1.0
    dinv = lax.rsqrt(deg)
    h1 = jnp.dot(x_ref[...], w1_ref[...], preferred_element_type=jnp.float32)
    g1_ref[...] = h1 * dinv
    dinv_ref[...] = dinv


_tc1 = pl.pallas_call(
    _tc1_body,
    out_shape=(
        jax.ShapeDtypeStruct((N, H), jnp.float32),
        jax.ShapeDtypeStruct((N, 1), jnp.float32),
    ),
)


def _tc2_body(aggp_ref, g1_ref, dinv_ref, b1_ref, gamma_ref, beta_ref,
              w2_ref, g2_ref):
    aggp = aggp_ref[...]
    agg = aggp[0, :N, :] + aggp[1, :N, :]
    dinv = dinv_ref[...]
    c1 = dinv * (agg + g1_ref[...]) + b1_ref[...]
    mean = jnp.mean(c1, axis=0, keepdims=True)
    var = jnp.mean((c1 - mean) * (c1 - mean), axis=0, keepdims=True)
    hb = (c1 - mean) * lax.rsqrt(var + 1e-5) * gamma_ref[...] + beta_ref[...]
    hr = jnp.maximum(hb, 0.0)
    t2 = jnp.dot(hr, w2_ref[...], preferred_element_type=jnp.float32)
    g2_ref[...] = t2 * dinv


_tc2 = pl.pallas_call(
    _tc2_body,
    out_shape=jax.ShapeDtypeStruct((N, H), jnp.float32),
)


def _tc3_body(aggp_ref, g2_ref, dinv_ref, b2_ref, wc_ref, bc_ref, out_ref):
    aggp = aggp_ref[...]
    agg = aggp[0, :N, :] + aggp[1, :N, :]
    dinv = dinv_ref[...]
    c2 = dinv * (agg + g2_ref[...]) + b2_ref[...]
    out_ref[...] = (
        jnp.dot(c2, wc_ref[...], preferred_element_type=jnp.float32)
        + bc_ref[...])


_tc3 = pl.pallas_call(
    _tc3_body,
    out_shape=jax.ShapeDtypeStruct((N, C), jnp.float32),
)


def kernel(x, edge_index, W1, b1, gamma1, beta1, W2, b2, Wc, bc):
    src = edge_index[0]
    dst = edge_index[1]
    pad = EP - E
    srcp = jnp.concatenate([src, jnp.zeros((pad,), jnp.int32)])
    dstp = jnp.concatenate([dst, jnp.full((pad,), N, jnp.int32)])
    srcp = srcp.reshape(NC, NS, NCHUNK, CH)
    dstp = dstp.reshape(NC, NS, NCHUNK, CH)

    ones_ch = jnp.ones((CH, DEGW), jnp.float32)
    zeros_f = jnp.zeros((CH, D), jnp.float32)

    degp = _sc_deg(dstp, zeros_f, ones_ch)
    g1, dinv = _tc1(x, W1, degp)
    agg1 = _sc_agg(g1, srcp, dstp, zeros_f)
    g2 = _tc2(agg1, g1, dinv, b1.reshape(1, H), gamma1.reshape(1, H),
              beta1.reshape(1, H), W2)
    agg2 = _sc_agg(g2, srcp, dstp, zeros_f)
    out = _tc3(agg2, g2, dinv, b2.reshape(1, H), Wc, bc.reshape(1, C))
    return out
